# baseline (speedup 1.0000x reference)
import functools

import jax
import jax.numpy as jnp
import numpy as np
from jax import lax
from jax.experimental import pallas as pl
from jax.experimental.pallas import tpu as pltpu
from jax.experimental.pallas import tpu_sc as plsc

F32 = jnp.float32
I32 = jnp.int32
_SC_PARAMS = pltpu.CompilerParams(needs_layout_passes=False)
NC, NS = 2, 16
NW = NC * NS
STAGE = 640
SUB = 128
NSUB = STAGE // SUB


def _pin_body(x_ref, wp_ref, bp_ref, we_ref, be_ref, o_ref):
    h = jnp.tanh(jnp.dot(x_ref[...], wp_ref[...],
                         preferred_element_type=F32) + bp_ref[...])
    o_ref[...] = jnp.tanh(jnp.dot(h, we_ref[...],
                                  preferred_element_type=F32) + be_ref[...])


def _proj_body(x_ref, w1_ref, b1_ref, w2_ref, b2_ref, wc_ref, o_ref):
    t = jnp.tanh(jnp.dot(x_ref[...], w1_ref[...],
                         preferred_element_type=F32) + b1_ref[...])
    u = jnp.dot(t, w2_ref[...], preferred_element_type=F32) + b2_ref[...]
    o_ref[...] = jnp.dot(u, wc_ref[...], preferred_element_type=F32)


def _combine_body(ca0, ca1, ca2, ca3, a00, a01, a02, a03, a0d,
                  a10, a11, a12, a13, a1d, bd, ba,
                  o0, o1, o2, o3):
    rdeg = 1.0 / jnp.maximum(a0d[...] + a1d[...], 1.0)
    o0[...] = ca0[...] + (a00[...] + a10[...]) * rdeg + bd[0]
    o1[...] = ca1[...] + (a01[...] + a11[...]) * rdeg
    o2[...] = ca2[...] + (a02[...] + a12[...]) * rdeg + ba[0]
    o3[...] = ca3[...] + (a03[...] + a13[...]) * rdeg


def _readout_body(gf0, gs1, gf2, gs3, fc0, sc0, fc1, sc1, dis_ref, ang_ref):
    edis = jnp.exp(-2.0 + 15.0 * jnp.tanh(gf0[...] + gs1[...]))
    ang = jnp.tanh(gf2[...] + gs3[...]) * 4.0
    bx = (fc0[...] + sc0[...]) * 0.5
    by = (fc1[...] + sc1[...]) * 0.5
    t = ang * np.float32(np.pi)
    tmp = jnp.minimum(jnp.abs(bx / (jnp.cos(t) + 1e-4)),
                      jnp.abs(by / (jnp.sin(t) + 1e-4)))
    dis_ref[...] = edis + tmp
    ang_ref[...] = ang


def _tc_pin(pin_feat, W_pin, b_pin, W_ew, b_ew):
    E = pin_feat.shape[0]
    rows = E // 8
    xr = pin_feat.reshape(rows, 128)
    k = W_pin.shape[0]
    wb1 = jnp.zeros((128, 128), F32)
    wb2 = jnp.zeros((128, 8), F32)
    for j in range(8):
        wb1 = wb1.at[j * k:(j + 1) * k, j * k:(j + 1) * k].set(W_pin)
        wb2 = wb2.at[j * k:(j + 1) * k, j:j + 1].set(W_ew)
    bb1 = jnp.tile(b_pin, 8).reshape(1, 128)
    BP = 10000
    out = pl.pallas_call(
        _pin_body,
        grid=(rows // BP,),
        in_specs=[pl.BlockSpec((BP, 128), lambda i: (i, 0)),
                  pl.BlockSpec((128, 128), lambda i: (0, 0)),
                  pl.BlockSpec((1, 128), lambda i: (0, 0)),
                  pl.BlockSpec((128, 8), lambda i: (0, 0)),
                  pl.BlockSpec((1, 1), lambda i: (0, 0))],
        out_specs=pl.BlockSpec((BP, 8), lambda i: (i, 0)),
        out_shape=jax.ShapeDtypeStruct((rows, 8), F32),
    )(xr, wb1, bb1, wb2, b_ew.reshape(1, 1))
    return out


def _tc_proj(x, W1, b1, W2, b2, Wc, bp):
    n, k = x.shape
    h = W1.shape[1]
    c = Wc.shape[1]
    return pl.pallas_call(
        _proj_body,
        grid=(n // bp,),
        in_specs=[pl.BlockSpec((bp, k), lambda i: (i, 0)),
                  pl.BlockSpec((k, h), lambda i: (0, 0)),
                  pl.BlockSpec((1, h), lambda i: (0, 0)),
                  pl.BlockSpec((h, h), lambda i: (0, 0)),
                  pl.BlockSpec((1, h), lambda i: (0, 0)),
                  pl.BlockSpec((h, c), lambda i: (0, 0))],
        out_specs=pl.BlockSpec((bp, c), lambda i: (i, 0)),
        out_shape=jax.ShapeDtypeStruct((n, c), F32),
    )(x, W1, b1.reshape(1, h), W2, b2.reshape(1, h), Wc)


def _tc_combine(ca, aggs, b_dis, b_ang):
    n = ca[0].shape[0]
    vec = pl.BlockSpec((n,), lambda i: (i,))
    scl = pl.BlockSpec((1,), lambda i: (0,))
    return pl.pallas_call(
        _combine_body,
        grid=(1,),
        in_specs=[vec] * 14 + [scl, scl],
        out_specs=[vec] * 4,
        out_shape=[jax.ShapeDtypeStruct((n,), F32)] * 4,
    )(*ca, *aggs, b_dis, b_ang)


def _tc_readout(cols):
    e = cols[0].shape[0]
    bp = 8192
    vec = pl.BlockSpec((bp,), lambda i: (i,))
    return pl.pallas_call(
        _readout_body,
        grid=((e + bp - 1) // bp,),
        in_specs=[vec] * 8,
        out_specs=[vec, vec],
        out_shape=[jax.ShapeDtypeStruct((e,), F32),
                   jax.ShapeDtypeStruct((e,), F32)],
    )(*cols)


def _sc_scatter(net_flat, ew, src, dst, zeros):
    n_pins = src.shape[0]
    cpad = zeros.shape[0] * NS
    rpt = zeros.shape[0]
    stages = n_pins // STAGE
    iters = (stages + NW - 1) // NW
    mesh = plsc.VectorSubcoreMesh(core_axis_name="c", subcore_axis_name="s")
    ii = None

    @functools.partial(
        pl.kernel,
        out_type=jax.ShapeDtypeStruct((NC * 5 * cpad,), F32),
        mesh=mesh,
        compiler_params=_SC_PARAMS,
        scratch_types=[
            pltpu.VMEM((net_flat.shape[0],), F32),
            pltpu.VMEM((STAGE,), I32),
            pltpu.VMEM((STAGE,), F32),
            pltpu.VMEM((SUB,), I32),
            [pltpu.VMEM((SUB,), F32) for _ in range(5)],
            [pltpu.VMEM_SHARED((cpad,), F32) for _ in range(5)],
            pltpu.VMEM((rpt,), F32),
        ],
    )
    def k(net_h, ew_h, src_h, dst_h, z_h, out_h,
          net_v, src_v, ew_v, dst_v, colbs, aggs, bounce_v):
        cid = lax.axis_index("c")
        sid = lax.axis_index("s")
        wid = sid * NC + cid
        zoff = sid * rpt
        pltpu.sync_copy(z_h, bounce_v)
        for j in range(5):
            pltpu.sync_copy(bounce_v, aggs[j].at[pl.ds(zoff, rpt)])
        pltpu.sync_copy(net_h, net_v)
        ones16 = jnp.full((16,), 1.0, F32)
        for i in range(SUB // 16):
            colbs[4][pl.ds(i * 16, 16)] = ones16
        plsc.subcore_barrier()

        def stage_body(it, carry):
            g = it * NW + wid

            @pl.when(g < stages)
            def _():
                base = pl.multiple_of(g * STAGE, 8)
                pltpu.sync_copy(src_h.at[pl.ds(base, STAGE)], src_v)
                pltpu.sync_copy(ew_h.at[pl.ds(base, STAGE)], ew_v)
                for s in range(NSUB):
                    pltpu.sync_copy(dst_h.at[pl.ds(base + s * SUB, SUB)],
                                    dst_v)
                    for i in range(SUB // 16):
                        o = s * SUB + i * 16
                        s4 = src_v[pl.ds(o, 16)] * 4
                        w = ew_v[pl.ds(o, 16)]
                        for j in range(4):
                            m = plsc.load_gather(net_v, [s4 + j]) * w
                            colbs[j][pl.ds(i * 16, 16)] = m
                    for j in range(5):
                        pltpu.sync_copy(colbs[j], aggs[j].at[dst_v], add=True)
            return carry

        lax.fori_loop(0, iters, stage_body, 0)
        plsc.subcore_barrier()
        for j in range(5):
            pltpu.sync_copy(aggs[j].at[pl.ds(zoff, rpt)], bounce_v)
            ooff = (cid * 5 + j) * cpad + zoff
            pltpu.sync_copy(bounce_v, out_h.at[pl.ds(ooff, rpt)])

    return k(net_flat, ew, src, dst, zeros)


def _sc_gather(tabs, fathers, sons):
    n_pt = fathers.shape[0]
    n_tab = tabs[0].shape[0]
    stages = n_pt // STAGE
    iters = (stages + NW - 1) // NW
    mesh = plsc.VectorSubcoreMesh(core_axis_name="c", subcore_axis_name="s")
    passes = [(0, True), (1, False), (2, True), (3, False),
              (4, True), (4, False), (5, True), (5, False)]

    @functools.partial(
        pl.kernel,
        out_type=[jax.ShapeDtypeStruct((n_pt,), F32) for _ in range(8)],
        mesh=mesh,
        compiler_params=_SC_PARAMS,
        scratch_types=[
            pltpu.VMEM((n_tab,), F32),
            pltpu.VMEM((STAGE,), I32),
            pltpu.VMEM((STAGE,), F32),
        ],
    )
    def k(t0, t1, t2, t3, t4, t5, fa_h, so_h,
          o0, o1, o2, o3, o4, o5, o6, o7, tab_v, idx_v, out_v):
        cid = lax.axis_index("c")
        sid = lax.axis_index("s")
        wid = sid * NC + cid
        tabs_h = [t0, t1, t2, t3, t4, t5]
        outs_h = [o0, o1, o2, o3, o4, o5, o6, o7]
        prev_t = -1
        for p, (t, use_f) in enumerate(passes):
            if t != prev_t:
                pltpu.sync_copy(tabs_h[t], tab_v)
                prev_t = t
            idx_h = fa_h if use_f else so_h
            out_h = outs_h[p]

            def stage_body(it, carry):
                g = it * NW + wid

                @pl.when(g < stages)
                def _():
                    base = pl.multiple_of(g * STAGE, 8)
                    pltpu.sync_copy(idx_h.at[pl.ds(base, STAGE)], idx_v)
                    for i in range(STAGE // 16):
                        v = idx_v[pl.ds(i * 16, 16)]
                        out_v[pl.ds(i * 16, 16)] = plsc.load_gather(tab_v, [v])
                    pltpu.sync_copy(out_v, out_h.at[pl.ds(base, STAGE)])
                return carry

            lax.fori_loop(0, iters, stage_body, 0)

    return k(*tabs, fathers, sons)


def kernel(cell_feat, net_feat, pin_feat, cell_size,
           pinned_src, pinned_dst, fathers, sons,
           W_cell, b_cell, W_net, b_net, W_pin, b_pin,
           W_ew, b_ew, W_self, W_neigh, b_sage,
           W_dis, b_dis, W_ang, b_ang):
    n_cells = cell_feat.shape[0]
    hc = W_cell.shape[1]
    Wcat = jnp.concatenate([W_dis[:hc], W_dis[hc:], W_ang[:hc], W_ang[hc:]],
                           axis=1)

    ew = _tc_pin(pin_feat, W_pin, b_pin, W_ew, b_ew).reshape(-1)
    cellA = _tc_proj(cell_feat, W_cell, b_cell, W_self, b_sage, Wcat, 5000)
    net4 = _tc_proj(net_feat, W_net, b_net, W_neigh,
                    jnp.zeros_like(b_sage), Wcat, 2000)

    rpt = ((n_cells + NS * 8 - 1) // (NS * 8)) * 8
    cpad = rpt * NS
    agg = _sc_scatter(net4.reshape(-1), ew, pinned_src, pinned_dst,
                      jnp.zeros((rpt,), F32))

    ca = [cellA[:, j] for j in range(4)]
    aggs = [agg[(c * 5 + j) * cpad:(c * 5 + j) * cpad + n_cells]
            for c in range(NC) for j in range(5)]
    scal = _tc_combine(ca, aggs, b_dis, b_ang)

    tabs = list(scal) + [cell_size[:, 0], cell_size[:, 1]]
    cols = _sc_gather(tabs, fathers, sons)
    edge_dis, edge_angle = _tc_readout(cols)
    return (edge_dis, edge_angle)


# speedup vs baseline: 1.1876x; 1.1876x over previous
import functools

import jax
import jax.numpy as jnp
import numpy as np
from jax import lax
from jax.experimental import pallas as pl
from jax.experimental.pallas import tpu as pltpu
from jax.experimental.pallas import tpu_sc as plsc

F32 = jnp.float32
I32 = jnp.int32
_SC_PARAMS = pltpu.CompilerParams(needs_layout_passes=False)
NC, NS = 2, 16
NW = NC * NS
STAGE = 640
SUB = 128
NSUB = STAGE // SUB


def _pin_body(x_ref, wp_ref, bp_ref, we_ref, be_ref, o_ref):
    h = jnp.tanh(jnp.dot(x_ref[...], wp_ref[...],
                         preferred_element_type=F32) + bp_ref[...])
    o_ref[...] = jnp.tanh(jnp.dot(h, we_ref[...],
                                  preferred_element_type=F32) + be_ref[...])


def _proj_body(x_ref, w1_ref, b1_ref, w2_ref, b2_ref, wc_ref, o_ref):
    t = jnp.tanh(jnp.dot(x_ref[...], w1_ref[...],
                         preferred_element_type=F32) + b1_ref[...])
    u = jnp.dot(t, w2_ref[...], preferred_element_type=F32) + b2_ref[...]
    o_ref[...] = jnp.dot(u, wc_ref[...], preferred_element_type=F32)


def _combine_body(*refs):
    nag = 5 * 6
    ca = refs[0:4]
    ag = refs[4:4 + nag]
    bd, ba = refs[4 + nag], refs[5 + nag]
    outs = refs[6 + nag:]
    cols = [sum(ag[j * 6 + r][...] for r in range(6)) for j in range(5)]
    rdeg = 1.0 / jnp.maximum(cols[4], 1.0)
    bias = [bd[0], 0.0, ba[0], 0.0]
    for j in range(4):
        outs[j][...] = ca[j][...] + cols[j] * rdeg + bias[j]


def _readout_body(gf0, gs1, gf2, gs3, fc0, sc0, fc1, sc1, dis_ref, ang_ref):
    edis = jnp.exp(-2.0 + 15.0 * jnp.tanh(gf0[...] + gs1[...]))
    ang = jnp.tanh(gf2[...] + gs3[...]) * 4.0
    bx = (fc0[...] + sc0[...]) * 0.5
    by = (fc1[...] + sc1[...]) * 0.5
    t = ang * np.float32(np.pi)
    tmp = jnp.minimum(jnp.abs(bx / (jnp.cos(t) + 1e-4)),
                      jnp.abs(by / (jnp.sin(t) + 1e-4)))
    dis_ref[...] = edis + tmp
    ang_ref[...] = ang


def _tc_pin(pin_feat, W_pin, b_pin, W_ew, b_ew):
    E = pin_feat.shape[0]
    rows = E // 8
    xr = pin_feat.reshape(rows, 128)
    k = W_pin.shape[0]
    wb1 = jnp.zeros((128, 128), F32)
    wb2 = jnp.zeros((128, 8), F32)
    for j in range(8):
        wb1 = wb1.at[j * k:(j + 1) * k, j * k:(j + 1) * k].set(W_pin)
        wb2 = wb2.at[j * k:(j + 1) * k, j:j + 1].set(W_ew)
    bb1 = jnp.tile(b_pin, 8).reshape(1, 128)
    BP = 10000
    out = pl.pallas_call(
        _pin_body,
        grid=(rows // BP,),
        in_specs=[pl.BlockSpec((BP, 128), lambda i: (i, 0)),
                  pl.BlockSpec((128, 128), lambda i: (0, 0)),
                  pl.BlockSpec((1, 128), lambda i: (0, 0)),
                  pl.BlockSpec((128, 8), lambda i: (0, 0)),
                  pl.BlockSpec((1, 1), lambda i: (0, 0))],
        out_specs=pl.BlockSpec((BP, 8), lambda i: (i, 0)),
        out_shape=jax.ShapeDtypeStruct((rows, 8), F32),
    )(xr, wb1, bb1, wb2, b_ew.reshape(1, 1))
    return out


def _tc_proj(x, W1, b1, W2, b2, Wc, bp):
    n, k = x.shape
    h = W1.shape[1]
    c = Wc.shape[1]
    return pl.pallas_call(
        _proj_body,
        grid=(n // bp,),
        in_specs=[pl.BlockSpec((bp, k), lambda i: (i, 0)),
                  pl.BlockSpec((k, h), lambda i: (0, 0)),
                  pl.BlockSpec((1, h), lambda i: (0, 0)),
                  pl.BlockSpec((h, h), lambda i: (0, 0)),
                  pl.BlockSpec((1, h), lambda i: (0, 0)),
                  pl.BlockSpec((h, c), lambda i: (0, 0))],
        out_specs=pl.BlockSpec((bp, c), lambda i: (i, 0)),
        out_shape=jax.ShapeDtypeStruct((n, c), F32),
    )(x, W1, b1.reshape(1, h), W2, b2.reshape(1, h), Wc)


def _tc_combine(ca, aggs, b_dis, b_ang):
    n = ca[0].shape[0]
    vec = pl.BlockSpec((n,), lambda i: (i,))
    scl = pl.BlockSpec((1,), lambda i: (0,))
    return pl.pallas_call(
        _combine_body,
        grid=(1,),
        in_specs=[vec] * 34 + [scl, scl],
        out_specs=[vec] * 4,
        out_shape=[jax.ShapeDtypeStruct((n,), F32)] * 4,
    )(*ca, *aggs, b_dis, b_ang)


def _tc_readout(cols):
    e = cols[0].shape[0]
    bp = 8192
    vec = pl.BlockSpec((bp,), lambda i: (i,))
    return pl.pallas_call(
        _readout_body,
        grid=((e + bp - 1) // bp,),
        in_specs=[vec] * 8,
        out_specs=[vec, vec],
        out_shape=[jax.ShapeDtypeStruct((e,), F32),
                   jax.ShapeDtypeStruct((e,), F32)],
    )(*cols)


SSTAGE = 1280
RANGES = 6


def _sc_scatter(net_flat, ew, src, dst, zeros):
    n_pins = src.shape[0]
    cpad = zeros.shape[0]
    stages = n_pins // SSTAGE
    iters2 = (stages + 2 * RANGES - 1) // (2 * RANGES)
    mesh = plsc.VectorSubcoreMesh(core_axis_name="c", subcore_axis_name="s")

    @functools.partial(
        pl.kernel,
        out_type=jax.ShapeDtypeStruct((5 * RANGES * cpad,), F32),
        mesh=mesh,
        compiler_params=_SC_PARAMS,
        scratch_types=[
            pltpu.VMEM((net_flat.shape[0],), F32),
            [pltpu.VMEM((SSTAGE,), I32) for _ in range(2)],
            [pltpu.VMEM((SSTAGE,), F32) for _ in range(2)],
            [pltpu.VMEM((SSTAGE,), I32) for _ in range(2)],
            pltpu.VMEM((cpad,), F32),
            [pltpu.SemaphoreType.DMA for _ in range(2)],
        ],
    )
    def k(net_h, ew_h, src_h, dst_h, z_h, out_h,
          net_v, srcs, ews, dsts, acc_v, sems):
        cid = lax.axis_index("c")
        sid = lax.axis_index("s")
        wid = sid * NC + cid
        j = wid // RANGES
        r = wid % RANGES
        active = wid < 5 * RANGES
        is_msg = active & (j < 4)
        my_stages = (stages - r + RANGES - 1) // RANGES

        @pl.when(active)
        def _():
            pltpu.sync_copy(z_h, acc_v)

        @pl.when(is_msg)
        def _():
            pltpu.sync_copy(net_h, net_v)

        def fire(kk, b):
            g = kk * RANGES + r
            base = pl.multiple_of(g * SSTAGE, 8)
            pltpu.async_copy(dst_h.at[pl.ds(base, SSTAGE)], dsts[b], sems[b])

            @pl.when(j < 4)
            def _():
                pltpu.async_copy(src_h.at[pl.ds(base, SSTAGE)], srcs[b],
                                 sems[b])
                pltpu.async_copy(ew_h.at[pl.ds(base, SSTAGE)], ews[b],
                                 sems[b])

        def wait(b):
            pltpu.make_async_copy(dst_h.at[pl.ds(0, SSTAGE)], dsts[b],
                                  sems[b]).wait()

            @pl.when(j < 4)
            def _():
                pltpu.make_async_copy(src_h.at[pl.ds(0, SSTAGE)], srcs[b],
                                      sems[b]).wait()
                pltpu.make_async_copy(ew_h.at[pl.ds(0, SSTAGE)], ews[b],
                                      sems[b]).wait()

        @pl.when(active)
        def _():
            fire(0, 0)

        ones16 = jnp.full((16,), 1.0, F32)

        def body2(it, carry):
            for b in range(2):
                kk = it * 2 + b

                @pl.when(active & (kk < my_stages))
                def _(kk=kk, b=b):
                    wait(b)

                    @pl.when(kk + 1 < my_stages)
                    def _():
                        fire(kk + 1, 1 - b)

                    @pl.when(j < 4)
                    def _():
                        for grp in range(SSTAGE // 16):
                            o = grp * 16
                            d16 = dsts[b][pl.ds(o, 16)]
                            s4 = srcs[b][pl.ds(o, 16)] * 4 + j
                            w = ews[b][pl.ds(o, 16)]
                            m = plsc.load_gather(net_v, [s4]) * w
                            plsc.addupdate_scatter(acc_v, [d16], m)

                    @pl.when(j == 4)
                    def _():
                        for grp in range(SSTAGE // 16):
                            o = grp * 16
                            d16 = dsts[b][pl.ds(o, 16)]
                            plsc.addupdate_scatter(acc_v, [d16], ones16)
            return carry

        lax.fori_loop(0, iters2, body2, 0)

        @pl.when(active)
        def _():
            pltpu.sync_copy(acc_v, out_h.at[pl.ds(wid * cpad, cpad)])

    return k(net_flat, ew, src, dst, zeros)


def _sc_gather(tabs, fathers, sons):
    n_pt = fathers.shape[0]
    n_tab = tabs[0].shape[0]
    stages = n_pt // STAGE
    iters = (stages + NW - 1) // NW
    mesh = plsc.VectorSubcoreMesh(core_axis_name="c", subcore_axis_name="s")
    passes = [(0, True), (1, False), (2, True), (3, False),
              (4, True), (4, False), (5, True), (5, False)]

    @functools.partial(
        pl.kernel,
        out_type=[jax.ShapeDtypeStruct((n_pt,), F32) for _ in range(8)],
        mesh=mesh,
        compiler_params=_SC_PARAMS,
        scratch_types=[
            pltpu.VMEM((n_tab,), F32),
            pltpu.VMEM((STAGE,), I32),
            pltpu.VMEM((STAGE,), F32),
        ],
    )
    def k(t0, t1, t2, t3, t4, t5, fa_h, so_h,
          o0, o1, o2, o3, o4, o5, o6, o7, tab_v, idx_v, out_v):
        cid = lax.axis_index("c")
        sid = lax.axis_index("s")
        wid = sid * NC + cid
        tabs_h = [t0, t1, t2, t3, t4, t5]
        outs_h = [o0, o1, o2, o3, o4, o5, o6, o7]
        prev_t = -1
        for p, (t, use_f) in enumerate(passes):
            if t != prev_t:
                pltpu.sync_copy(tabs_h[t], tab_v)
                prev_t = t
            idx_h = fa_h if use_f else so_h
            out_h = outs_h[p]

            def stage_body(it, carry):
                g = it * NW + wid

                @pl.when(g < stages)
                def _():
                    base = pl.multiple_of(g * STAGE, 8)
                    pltpu.sync_copy(idx_h.at[pl.ds(base, STAGE)], idx_v)
                    for i in range(STAGE // 16):
                        v = idx_v[pl.ds(i * 16, 16)]
                        out_v[pl.ds(i * 16, 16)] = plsc.load_gather(tab_v, [v])
                    pltpu.sync_copy(out_v, out_h.at[pl.ds(base, STAGE)])
                return carry

            lax.fori_loop(0, iters, stage_body, 0)

    return k(*tabs, fathers, sons)


def kernel(cell_feat, net_feat, pin_feat, cell_size,
           pinned_src, pinned_dst, fathers, sons,
           W_cell, b_cell, W_net, b_net, W_pin, b_pin,
           W_ew, b_ew, W_self, W_neigh, b_sage,
           W_dis, b_dis, W_ang, b_ang):
    n_cells = cell_feat.shape[0]
    hc = W_cell.shape[1]
    Wcat = jnp.concatenate([W_dis[:hc], W_dis[hc:], W_ang[:hc], W_ang[hc:]],
                           axis=1)

    ew = _tc_pin(pin_feat, W_pin, b_pin, W_ew, b_ew).reshape(-1)
    cellA = _tc_proj(cell_feat, W_cell, b_cell, W_self, b_sage, Wcat, 5000)
    net4 = _tc_proj(net_feat, W_net, b_net, W_neigh,
                    jnp.zeros_like(b_sage), Wcat, 2000)

    cpad = ((n_cells + 7) // 8) * 8
    agg = _sc_scatter(net4.reshape(-1), ew, pinned_src, pinned_dst,
                      jnp.zeros((cpad,), F32))

    ca = [cellA[:, j] for j in range(4)]
    aggs = [agg[w * cpad:w * cpad + n_cells] for w in range(5 * RANGES)]
    scal = _tc_combine(ca, aggs, b_dis, b_ang)

    tabs = list(scal) + [cell_size[:, 0], cell_size[:, 1]]
    cols = _sc_gather(tabs, fathers, sons)
    edge_dis, edge_angle = _tc_readout(cols)
    return (edge_dis, edge_angle)


# speedup vs baseline: 1.3128x; 1.1055x over previous
import functools

import jax
import jax.numpy as jnp
import numpy as np
from jax import lax
from jax.experimental import pallas as pl
from jax.experimental.pallas import tpu as pltpu
from jax.experimental.pallas import tpu_sc as plsc

F32 = jnp.float32
I32 = jnp.int32
_SC_PARAMS = pltpu.CompilerParams(needs_layout_passes=False)
NC, NS = 2, 16
NW = NC * NS
STAGE = 640
SUB = 128
NSUB = STAGE // SUB


def _pin_body(x_ref, wp_ref, bp_ref, we_ref, be_ref, o_ref):
    h = jnp.tanh(jnp.dot(x_ref[...], wp_ref[...],
                         preferred_element_type=F32) + bp_ref[...])
    o_ref[...] = jnp.tanh(jnp.dot(h, we_ref[...],
                                  preferred_element_type=F32) + be_ref[...])


def _proj_body(x_ref, w1_ref, b1_ref, w2_ref, b2_ref, wc_ref, o_ref):
    t = jnp.tanh(jnp.dot(x_ref[...], w1_ref[...],
                         preferred_element_type=F32) + b1_ref[...])
    u = jnp.dot(t, w2_ref[...], preferred_element_type=F32) + b2_ref[...]
    o_ref[...] = jnp.dot(u, wc_ref[...], preferred_element_type=F32)


def _combine_body(*refs):
    nag = 5 * 6
    ca = refs[0:4]
    ag = refs[4:4 + nag]
    bd, ba = refs[4 + nag], refs[5 + nag]
    outs = refs[6 + nag:]
    cols = [sum(ag[j * 6 + r][...] for r in range(6)) for j in range(5)]
    rdeg = 1.0 / jnp.maximum(cols[4], 1.0)
    bias = [bd[0], 0.0, ba[0], 0.0]
    for j in range(4):
        outs[j][...] = ca[j][...] + cols[j] * rdeg + bias[j]


def _readout_body(gf0, gs1, gf2, gs3, fc0, sc0, fc1, sc1, dis_ref, ang_ref):
    edis = jnp.exp(-2.0 + 15.0 * jnp.tanh(gf0[...] + gs1[...]))
    ang = jnp.tanh(gf2[...] + gs3[...]) * 4.0
    bx = (fc0[...] + sc0[...]) * 0.5
    by = (fc1[...] + sc1[...]) * 0.5
    t = ang * np.float32(np.pi)
    tmp = jnp.minimum(jnp.abs(bx / (jnp.cos(t) + 1e-4)),
                      jnp.abs(by / (jnp.sin(t) + 1e-4)))
    dis_ref[...] = edis + tmp
    ang_ref[...] = ang


def _tc_pin(pin_feat, W_pin, b_pin, W_ew, b_ew):
    E = pin_feat.shape[0]
    rows = E // 8
    xr = pin_feat.reshape(rows, 128)
    k = W_pin.shape[0]
    wb1 = jnp.zeros((128, 128), F32)
    wb2 = jnp.zeros((128, 8), F32)
    for j in range(8):
        wb1 = wb1.at[j * k:(j + 1) * k, j * k:(j + 1) * k].set(W_pin)
        wb2 = wb2.at[j * k:(j + 1) * k, j:j + 1].set(W_ew)
    bb1 = jnp.tile(b_pin, 8).reshape(1, 128)
    BP = 10000
    out = pl.pallas_call(
        _pin_body,
        grid=(rows // BP,),
        in_specs=[pl.BlockSpec((BP, 128), lambda i: (i, 0)),
                  pl.BlockSpec((128, 128), lambda i: (0, 0)),
                  pl.BlockSpec((1, 128), lambda i: (0, 0)),
                  pl.BlockSpec((128, 8), lambda i: (0, 0)),
                  pl.BlockSpec((1, 1), lambda i: (0, 0))],
        out_specs=pl.BlockSpec((BP, 8), lambda i: (i, 0)),
        out_shape=jax.ShapeDtypeStruct((rows, 8), F32),
    )(xr, wb1, bb1, wb2, b_ew.reshape(1, 1))
    return out


def _tc_proj(x, W1, b1, W2, b2, Wc, bp):
    n, k = x.shape
    h = W1.shape[1]
    c = Wc.shape[1]
    return pl.pallas_call(
        _proj_body,
        grid=(n // bp,),
        in_specs=[pl.BlockSpec((bp, k), lambda i: (i, 0)),
                  pl.BlockSpec((k, h), lambda i: (0, 0)),
                  pl.BlockSpec((1, h), lambda i: (0, 0)),
                  pl.BlockSpec((h, h), lambda i: (0, 0)),
                  pl.BlockSpec((1, h), lambda i: (0, 0)),
                  pl.BlockSpec((h, c), lambda i: (0, 0))],
        out_specs=pl.BlockSpec((bp, c), lambda i: (i, 0)),
        out_shape=jax.ShapeDtypeStruct((n, c), F32),
    )(x, W1, b1.reshape(1, h), W2, b2.reshape(1, h), Wc)


def _tc_combine(ca, aggs, b_dis, b_ang):
    n = ca[0].shape[0]
    vec = pl.BlockSpec((n,), lambda i: (i,))
    scl = pl.BlockSpec((1,), lambda i: (0,))
    return pl.pallas_call(
        _combine_body,
        grid=(1,),
        in_specs=[vec] * 34 + [scl, scl],
        out_specs=[vec] * 4,
        out_shape=[jax.ShapeDtypeStruct((n,), F32)] * 4,
    )(*ca, *aggs, b_dis, b_ang)


def _tc_readout(cols):
    e = cols[0].shape[0]
    bp = 8192
    vec = pl.BlockSpec((bp,), lambda i: (i,))
    return pl.pallas_call(
        _readout_body,
        grid=((e + bp - 1) // bp,),
        in_specs=[vec] * 8,
        out_specs=[vec, vec],
        out_shape=[jax.ShapeDtypeStruct((e,), F32),
                   jax.ShapeDtypeStruct((e,), F32)],
    )(*cols)


SSTAGE = 1280
RANGES = 6


def _sc_scatter(net_flat, ew, src, dst, zeros):
    n_pins = src.shape[0]
    cpad = zeros.shape[0]
    stages = n_pins // SSTAGE
    iters2 = (stages + 2 * RANGES - 1) // (2 * RANGES)
    mesh = plsc.VectorSubcoreMesh(core_axis_name="c", subcore_axis_name="s")

    @functools.partial(
        pl.kernel,
        out_type=jax.ShapeDtypeStruct((5 * RANGES * cpad,), F32),
        mesh=mesh,
        compiler_params=_SC_PARAMS,
        scratch_types=[
            pltpu.VMEM((net_flat.shape[0],), F32),
            [pltpu.VMEM((SSTAGE,), I32) for _ in range(2)],
            [pltpu.VMEM((SSTAGE,), F32) for _ in range(2)],
            [pltpu.VMEM((SSTAGE,), I32) for _ in range(2)],
            pltpu.VMEM((cpad,), F32),
            [pltpu.SemaphoreType.DMA for _ in range(2)],
        ],
    )
    def k(net_h, ew_h, src_h, dst_h, z_h, out_h,
          net_v, srcs, ews, dsts, acc_v, sems):
        cid = lax.axis_index("c")
        sid = lax.axis_index("s")
        wid = sid * NC + cid
        j = wid // RANGES
        r = wid % RANGES
        active = wid < 5 * RANGES
        is_msg = active & (j < 4)
        my_stages = (stages - r + RANGES - 1) // RANGES

        @pl.when(active)
        def _():
            pltpu.sync_copy(z_h, acc_v)

        @pl.when(is_msg)
        def _():
            pltpu.sync_copy(net_h, net_v)

        def fire(kk, b):
            g = kk * RANGES + r
            base = pl.multiple_of(g * SSTAGE, 8)
            pltpu.async_copy(dst_h.at[pl.ds(base, SSTAGE)], dsts[b], sems[b])

            @pl.when(j < 4)
            def _():
                pltpu.async_copy(src_h.at[pl.ds(base, SSTAGE)], srcs[b],
                                 sems[b])
                pltpu.async_copy(ew_h.at[pl.ds(base, SSTAGE)], ews[b],
                                 sems[b])

        def wait(b):
            pltpu.make_async_copy(dst_h.at[pl.ds(0, SSTAGE)], dsts[b],
                                  sems[b]).wait()

            @pl.when(j < 4)
            def _():
                pltpu.make_async_copy(src_h.at[pl.ds(0, SSTAGE)], srcs[b],
                                      sems[b]).wait()
                pltpu.make_async_copy(ew_h.at[pl.ds(0, SSTAGE)], ews[b],
                                      sems[b]).wait()

        @pl.when(active)
        def _():
            fire(0, 0)

        ones16 = jnp.full((16,), 1.0, F32)

        def body2(it, carry):
            for b in range(2):
                kk = it * 2 + b

                @pl.when(active & (kk < my_stages))
                def _(kk=kk, b=b):
                    wait(b)

                    @pl.when(kk + 1 < my_stages)
                    def _():
                        fire(kk + 1, 1 - b)

                    @pl.when(j < 4)
                    def _():
                        for grp in range(SSTAGE // 16):
                            o = grp * 16
                            d16 = dsts[b][pl.ds(o, 16)]
                            s4 = srcs[b][pl.ds(o, 16)] * 4 + j
                            w = ews[b][pl.ds(o, 16)]
                            m = plsc.load_gather(net_v, [s4]) * w
                            plsc.addupdate_scatter(acc_v, [d16], m)

                    @pl.when(j == 4)
                    def _():
                        for grp in range(SSTAGE // 16):
                            o = grp * 16
                            d16 = dsts[b][pl.ds(o, 16)]
                            plsc.addupdate_scatter(acc_v, [d16], ones16)
            return carry

        lax.fori_loop(0, iters2, body2, 0)

        @pl.when(active)
        def _():
            pltpu.sync_copy(acc_v, out_h.at[pl.ds(wid * cpad, cpad)])

    return k(net_flat, ew, src, dst, zeros)


GSTAGE = 1600


def _sc_gather(tabs, fathers, sons):
    n_pt = fathers.shape[0]
    n_tab = tabs[0].shape[0]
    stages = n_pt // GSTAGE
    iters2 = (stages // NW + 2) // 2
    mesh = plsc.VectorSubcoreMesh(core_axis_name="c", subcore_axis_name="s")
    passes = [(0, True), (1, False), (2, True), (3, False),
              (4, True), (4, False), (5, True), (5, False)]

    @functools.partial(
        pl.kernel,
        out_type=[jax.ShapeDtypeStruct((n_pt,), F32) for _ in range(8)],
        mesh=mesh,
        compiler_params=_SC_PARAMS,
        scratch_types=[
            pltpu.VMEM((n_tab,), F32),
            [pltpu.VMEM((GSTAGE,), I32) for _ in range(2)],
            [pltpu.VMEM((GSTAGE,), F32) for _ in range(2)],
            [pltpu.SemaphoreType.DMA for _ in range(2)],
            [pltpu.SemaphoreType.DMA for _ in range(2)],
        ],
    )
    def k(t0, t1, t2, t3, t4, t5, fa_h, so_h,
          o0, o1, o2, o3, o4, o5, o6, o7,
          tab_v, idxs, outs, isems, osems):
        cid = lax.axis_index("c")
        sid = lax.axis_index("s")
        wid = sid * NC + cid
        tabs_h = [t0, t1, t2, t3, t4, t5]
        outs_h = [o0, o1, o2, o3, o4, o5, o6, o7]
        my_stages = (stages - wid + NW - 1) // NW
        prev_t = -1
        for p, (t, use_f) in enumerate(passes):
            if t != prev_t:
                pltpu.sync_copy(tabs_h[t], tab_v)
                prev_t = t
            idx_h = fa_h if use_f else so_h
            out_h = outs_h[p]

            def fire_idx(kk, b, idx_h=idx_h):
                base = pl.multiple_of((kk * NW + wid) * GSTAGE, 8)
                pltpu.async_copy(idx_h.at[pl.ds(base, GSTAGE)], idxs[b],
                                 isems[b])

            def wait_idx(b, idx_h=idx_h):
                pltpu.make_async_copy(idx_h.at[pl.ds(0, GSTAGE)], idxs[b],
                                      isems[b]).wait()

            def fire_out(kk, b, out_h=out_h):
                base = pl.multiple_of((kk * NW + wid) * GSTAGE, 8)
                pltpu.async_copy(outs[b], out_h.at[pl.ds(base, GSTAGE)],
                                 osems[b])

            def drain_out(b, out_h=out_h):
                pltpu.make_async_copy(outs[b], out_h.at[pl.ds(0, GSTAGE)],
                                      osems[b]).wait()

            fire_idx(0, 0)

            def body2(it, carry):
                for b in range(2):
                    kk = it * 2 + b

                    @pl.when(kk < my_stages)
                    def _(kk=kk, b=b):
                        wait_idx(b)

                        @pl.when(kk + 1 < my_stages)
                        def _():
                            fire_idx(kk + 1, 1 - b)

                        @pl.when(kk >= 2)
                        def _():
                            drain_out(b)
                        for i in range(GSTAGE // 16):
                            v = idxs[b][pl.ds(i * 16, 16)]
                            outs[b][pl.ds(i * 16, 16)] = (
                                plsc.load_gather(tab_v, [v]))
                        fire_out(kk, b)
                return carry

            lax.fori_loop(0, iters2, body2, 0)
            drain_out(0)
            drain_out(1)

    return k(*tabs, fathers, sons)


def kernel(cell_feat, net_feat, pin_feat, cell_size,
           pinned_src, pinned_dst, fathers, sons,
           W_cell, b_cell, W_net, b_net, W_pin, b_pin,
           W_ew, b_ew, W_self, W_neigh, b_sage,
           W_dis, b_dis, W_ang, b_ang):
    n_cells = cell_feat.shape[0]
    hc = W_cell.shape[1]
    Wcat = jnp.concatenate([W_dis[:hc], W_dis[hc:], W_ang[:hc], W_ang[hc:]],
                           axis=1)

    ew = _tc_pin(pin_feat, W_pin, b_pin, W_ew, b_ew).reshape(-1)
    cellA = _tc_proj(cell_feat, W_cell, b_cell, W_self, b_sage, Wcat, 5000)
    net4 = _tc_proj(net_feat, W_net, b_net, W_neigh,
                    jnp.zeros_like(b_sage), Wcat, 2000)

    cpad = ((n_cells + 7) // 8) * 8
    agg = _sc_scatter(net4.reshape(-1), ew, pinned_src, pinned_dst,
                      jnp.zeros((cpad,), F32))

    ca = [cellA[:, j] for j in range(4)]
    aggs = [agg[w * cpad:w * cpad + n_cells] for w in range(5 * RANGES)]
    scal = _tc_combine(ca, aggs, b_dis, b_ang)

    tabs = list(scal) + [cell_size[:, 0], cell_size[:, 1]]
    cols = _sc_gather(tabs, fathers, sons)
    edge_dis, edge_angle = _tc_readout(cols)
    return (edge_dis, edge_angle)


# speedup vs baseline: 1.3475x; 1.0264x over previous
import functools

import jax
import jax.numpy as jnp
import numpy as np
from jax import lax
from jax.experimental import pallas as pl
from jax.experimental.pallas import tpu as pltpu
from jax.experimental.pallas import tpu_sc as plsc

F32 = jnp.float32
I32 = jnp.int32
_SC_PARAMS = pltpu.CompilerParams(needs_layout_passes=False)
NC, NS = 2, 16
NW = NC * NS
STAGE = 640
SUB = 128
NSUB = STAGE // SUB


def _pin_body(x_ref, wp_ref, bp_ref, we_ref, be_ref, o_ref):
    h = jnp.tanh(jnp.dot(x_ref[...], wp_ref[...],
                         preferred_element_type=F32) + bp_ref[...])
    o_ref[...] = jnp.tanh(jnp.dot(h, we_ref[...],
                                  preferred_element_type=F32) + be_ref[...])


def _proj_body(x_ref, w1_ref, b1_ref, w2_ref, b2_ref, wc_ref, o_ref):
    t = jnp.tanh(jnp.dot(x_ref[...], w1_ref[...],
                         preferred_element_type=F32) + b1_ref[...])
    u = jnp.dot(t, w2_ref[...], preferred_element_type=F32) + b2_ref[...]
    o_ref[...] = jnp.dot(u, wc_ref[...], preferred_element_type=F32)


def _combine_body(*refs):
    nag = 4 * RMSG + RDEG
    ca = refs[0:4]
    ag = refs[4:4 + nag]
    bd, ba = refs[4 + nag], refs[5 + nag]
    outs = refs[6 + nag:]
    cols = [sum(ag[j * RMSG + r][...] for r in range(RMSG)) for j in range(4)]
    deg = sum(ag[4 * RMSG + r][...] for r in range(RDEG))
    rdeg = 1.0 / jnp.maximum(deg, 1.0)
    bias = [bd[0], 0.0, ba[0], 0.0]
    for j in range(4):
        outs[j][...] = ca[j][...] + cols[j] * rdeg + bias[j]


def _readout_body(gf0, gs1, gf2, gs3, fc0, sc0, fc1, sc1, dis_ref, ang_ref):
    edis = jnp.exp(-2.0 + 15.0 * jnp.tanh(gf0[...] + gs1[...]))
    ang = jnp.tanh(gf2[...] + gs3[...]) * 4.0
    bx = (fc0[...] + sc0[...]) * 0.5
    by = (fc1[...] + sc1[...]) * 0.5
    t = ang * np.float32(np.pi)
    tmp = jnp.minimum(jnp.abs(bx / (jnp.cos(t) + 1e-4)),
                      jnp.abs(by / (jnp.sin(t) + 1e-4)))
    dis_ref[...] = edis + tmp
    ang_ref[...] = ang


def _tc_pin(pin_feat, W_pin, b_pin, W_ew, b_ew):
    E = pin_feat.shape[0]
    rows = E // 8
    xr = pin_feat.reshape(rows, 128)
    k = W_pin.shape[0]
    wb1 = jnp.zeros((128, 128), F32)
    wb2 = jnp.zeros((128, 8), F32)
    for j in range(8):
        wb1 = wb1.at[j * k:(j + 1) * k, j * k:(j + 1) * k].set(W_pin)
        wb2 = wb2.at[j * k:(j + 1) * k, j:j + 1].set(W_ew)
    bb1 = jnp.tile(b_pin, 8).reshape(1, 128)
    BP = 10000
    out = pl.pallas_call(
        _pin_body,
        grid=(rows // BP,),
        in_specs=[pl.BlockSpec((BP, 128), lambda i: (i, 0)),
                  pl.BlockSpec((128, 128), lambda i: (0, 0)),
                  pl.BlockSpec((1, 128), lambda i: (0, 0)),
                  pl.BlockSpec((128, 8), lambda i: (0, 0)),
                  pl.BlockSpec((1, 1), lambda i: (0, 0))],
        out_specs=pl.BlockSpec((BP, 8), lambda i: (i, 0)),
        out_shape=jax.ShapeDtypeStruct((rows, 8), F32),
    )(xr, wb1, bb1, wb2, b_ew.reshape(1, 1))
    return out


def _tc_proj(x, W1, b1, W2, b2, Wc, bp):
    n, k = x.shape
    h = W1.shape[1]
    c = Wc.shape[1]
    return pl.pallas_call(
        _proj_body,
        grid=(n // bp,),
        in_specs=[pl.BlockSpec((bp, k), lambda i: (i, 0)),
                  pl.BlockSpec((k, h), lambda i: (0, 0)),
                  pl.BlockSpec((1, h), lambda i: (0, 0)),
                  pl.BlockSpec((h, h), lambda i: (0, 0)),
                  pl.BlockSpec((1, h), lambda i: (0, 0)),
                  pl.BlockSpec((h, c), lambda i: (0, 0))],
        out_specs=pl.BlockSpec((bp, c), lambda i: (i, 0)),
        out_shape=jax.ShapeDtypeStruct((n, c), F32),
    )(x, W1, b1.reshape(1, h), W2, b2.reshape(1, h), Wc)


def _tc_combine(ca, aggs, b_dis, b_ang):
    n = ca[0].shape[0]
    vec = pl.BlockSpec((n,), lambda i: (i,))
    scl = pl.BlockSpec((1,), lambda i: (0,))
    return pl.pallas_call(
        _combine_body,
        grid=(1,),
        in_specs=[vec] * 36 + [scl, scl],
        out_specs=[vec] * 4,
        out_shape=[jax.ShapeDtypeStruct((n,), F32)] * 4,
    )(*ca, *aggs, b_dis, b_ang)


def _tc_readout(cols):
    e = cols[0].shape[0]
    bp = 8192
    vec = pl.BlockSpec((bp,), lambda i: (i,))
    return pl.pallas_call(
        _readout_body,
        grid=((e + bp - 1) // bp,),
        in_specs=[vec] * 8,
        out_specs=[vec, vec],
        out_shape=[jax.ShapeDtypeStruct((e,), F32),
                   jax.ShapeDtypeStruct((e,), F32)],
    )(*cols)


SSTAGE = 1280
RMSG = 7
RDEG = 4


def _sc_scatter(net_flat, ew, src, dst, zeros):
    n_pins = src.shape[0]
    cpad = zeros.shape[0]
    stages = n_pins // SSTAGE
    iters2 = ((stages + RDEG - 1) // RDEG + 1) // 2 + 1
    mesh = plsc.VectorSubcoreMesh(core_axis_name="c", subcore_axis_name="s")

    @functools.partial(
        pl.kernel,
        out_type=jax.ShapeDtypeStruct((NW * cpad,), F32),
        mesh=mesh,
        compiler_params=_SC_PARAMS,
        scratch_types=[
            pltpu.VMEM((net_flat.shape[0],), F32),
            [pltpu.VMEM((SSTAGE,), I32) for _ in range(2)],
            [pltpu.VMEM((SSTAGE,), F32) for _ in range(2)],
            [pltpu.VMEM((SSTAGE,), I32) for _ in range(2)],
            pltpu.VMEM((cpad,), F32),
            [pltpu.SemaphoreType.DMA for _ in range(2)],
        ],
    )
    def k(net_h, ew_h, src_h, dst_h, z_h, out_h,
          net_v, srcs, ews, dsts, acc_v, sems):
        cid = lax.axis_index("c")
        sid = lax.axis_index("s")
        wid = sid * NC + cid
        is_msg = wid < 4 * RMSG
        j = jnp.where(is_msg, wid // RMSG, 4)
        r = jnp.where(is_msg, wid % RMSG, wid - 4 * RMSG)
        ranges = jnp.where(is_msg, RMSG, RDEG)
        my_stages = (stages - r + ranges - 1) // ranges

        pltpu.sync_copy(z_h, acc_v)

        @pl.when(is_msg)
        def _():
            pltpu.sync_copy(net_h, net_v)

        def fire(kk, b):
            g = kk * ranges + r
            base = pl.multiple_of(g * SSTAGE, 8)
            pltpu.async_copy(dst_h.at[pl.ds(base, SSTAGE)], dsts[b], sems[b])

            @pl.when(j < 4)
            def _():
                pltpu.async_copy(src_h.at[pl.ds(base, SSTAGE)], srcs[b],
                                 sems[b])
                pltpu.async_copy(ew_h.at[pl.ds(base, SSTAGE)], ews[b],
                                 sems[b])

        def wait(b):
            pltpu.make_async_copy(dst_h.at[pl.ds(0, SSTAGE)], dsts[b],
                                  sems[b]).wait()

            @pl.when(j < 4)
            def _():
                pltpu.make_async_copy(src_h.at[pl.ds(0, SSTAGE)], srcs[b],
                                      sems[b]).wait()
                pltpu.make_async_copy(ew_h.at[pl.ds(0, SSTAGE)], ews[b],
                                      sems[b]).wait()

        fire(0, 0)

        ones16 = jnp.full((16,), 1.0, F32)

        def body2(it, carry):
            for b in range(2):
                kk = it * 2 + b

                @pl.when(kk < my_stages)
                def _(kk=kk, b=b):
                    wait(b)

                    @pl.when(kk + 1 < my_stages)
                    def _():
                        fire(kk + 1, 1 - b)

                    @pl.when(j < 4)
                    def _():
                        for grp in range(SSTAGE // 16):
                            o = grp * 16
                            d16 = dsts[b][pl.ds(o, 16)]
                            s4 = srcs[b][pl.ds(o, 16)] * 4 + j
                            w = ews[b][pl.ds(o, 16)]
                            m = plsc.load_gather(net_v, [s4]) * w
                            plsc.addupdate_scatter(acc_v, [d16], m)

                    @pl.when(j == 4)
                    def _():
                        for grp in range(SSTAGE // 16):
                            o = grp * 16
                            d16 = dsts[b][pl.ds(o, 16)]
                            plsc.addupdate_scatter(acc_v, [d16], ones16)
            return carry

        lax.fori_loop(0, iters2, body2, 0)
        pltpu.sync_copy(acc_v, out_h.at[pl.ds(wid * cpad, cpad)])

    return k(net_flat, ew, src, dst, zeros)


GSTAGE = 1600


def _sc_gather(tabs, fathers, sons):
    n_pt = fathers.shape[0]
    n_tab = tabs[0].shape[0]
    stages = n_pt // GSTAGE
    iters2 = (stages // NW + 2) // 2
    mesh = plsc.VectorSubcoreMesh(core_axis_name="c", subcore_axis_name="s")
    passes = [(0, True), (1, False), (2, True), (3, False),
              (4, True), (4, False), (5, True), (5, False)]

    @functools.partial(
        pl.kernel,
        out_type=[jax.ShapeDtypeStruct((n_pt,), F32) for _ in range(8)],
        mesh=mesh,
        compiler_params=_SC_PARAMS,
        scratch_types=[
            [pltpu.VMEM((n_tab,), F32) for _ in range(2)],
            [pltpu.VMEM((GSTAGE,), I32) for _ in range(2)],
            [pltpu.VMEM((GSTAGE,), F32) for _ in range(2)],
            [pltpu.SemaphoreType.DMA for _ in range(2)],
            [pltpu.SemaphoreType.DMA for _ in range(2)],
            [pltpu.SemaphoreType.DMA for _ in range(2)],
        ],
    )
    def k(t0, t1, t2, t3, t4, t5, fa_h, so_h,
          o0, o1, o2, o3, o4, o5, o6, o7,
          tabs_v, idxs, outs, isems, osems, tsems):
        cid = lax.axis_index("c")
        sid = lax.axis_index("s")
        wid = sid * NC + cid
        tabs_h = [t0, t1, t2, t3, t4, t5]
        outs_h = [o0, o1, o2, o3, o4, o5, o6, o7]
        my_stages = (stages - wid + NW - 1) // NW

        def fire_tab(t, s):
            pltpu.async_copy(tabs_h[t], tabs_v[s], tsems[s])

        def wait_tab(t, s):
            pltpu.make_async_copy(tabs_h[t], tabs_v[s], tsems[s]).wait()

        fire_at = {0: [], 1: [(2, 0)], 2: [(3, 1)], 3: [(4, 0)], 4: [(5, 1)]}
        wait_at = {0: (0, 0), 1: (1, 1), 2: (2, 0), 3: (3, 1), 4: (4, 0),
                   6: (5, 1)}
        fire_tab(0, 0)
        fire_tab(1, 1)
        for p, (t, use_f) in enumerate(passes):
            for ft, fs in fire_at.get(p, []):
                fire_tab(ft, fs)
            if p in wait_at:
                wait_tab(*wait_at[p])
            tab_v = tabs_v[t % 2]
            idx_h = fa_h if use_f else so_h
            out_h = outs_h[p]

            def fire_idx(kk, b, idx_h=idx_h):
                base = pl.multiple_of((kk * NW + wid) * GSTAGE, 8)
                pltpu.async_copy(idx_h.at[pl.ds(base, GSTAGE)], idxs[b],
                                 isems[b])

            def wait_idx(b, idx_h=idx_h):
                pltpu.make_async_copy(idx_h.at[pl.ds(0, GSTAGE)], idxs[b],
                                      isems[b]).wait()

            def fire_out(kk, b, out_h=out_h):
                base = pl.multiple_of((kk * NW + wid) * GSTAGE, 8)
                pltpu.async_copy(outs[b], out_h.at[pl.ds(base, GSTAGE)],
                                 osems[b])

            def drain_out(b, out_h=out_h):
                pltpu.make_async_copy(outs[b], out_h.at[pl.ds(0, GSTAGE)],
                                      osems[b]).wait()

            fire_idx(0, 0)

            def body2(it, carry):
                for b in range(2):
                    kk = it * 2 + b

                    @pl.when(kk < my_stages)
                    def _(kk=kk, b=b):
                        wait_idx(b)

                        @pl.when(kk + 1 < my_stages)
                        def _():
                            fire_idx(kk + 1, 1 - b)

                        @pl.when(kk >= 2)
                        def _():
                            drain_out(b)
                        for i in range(GSTAGE // 16):
                            v = idxs[b][pl.ds(i * 16, 16)]
                            outs[b][pl.ds(i * 16, 16)] = (
                                plsc.load_gather(tab_v, [v]))
                        fire_out(kk, b)
                return carry

            lax.fori_loop(0, iters2, body2, 0)
            drain_out(0)
            drain_out(1)

    return k(*tabs, fathers, sons)


def kernel(cell_feat, net_feat, pin_feat, cell_size,
           pinned_src, pinned_dst, fathers, sons,
           W_cell, b_cell, W_net, b_net, W_pin, b_pin,
           W_ew, b_ew, W_self, W_neigh, b_sage,
           W_dis, b_dis, W_ang, b_ang):
    n_cells = cell_feat.shape[0]
    hc = W_cell.shape[1]
    Wcat = jnp.concatenate([W_dis[:hc], W_dis[hc:], W_ang[:hc], W_ang[hc:]],
                           axis=1)

    ew = _tc_pin(pin_feat, W_pin, b_pin, W_ew, b_ew).reshape(-1)
    cellA = _tc_proj(cell_feat, W_cell, b_cell, W_self, b_sage, Wcat, 5000)
    net4 = _tc_proj(net_feat, W_net, b_net, W_neigh,
                    jnp.zeros_like(b_sage), Wcat, 2000)

    cpad = ((n_cells + 7) // 8) * 8
    agg = _sc_scatter(net4.reshape(-1), ew, pinned_src, pinned_dst,
                      jnp.zeros((cpad,), F32))

    ca = [cellA[:, j] for j in range(4)]
    aggs = [agg[w * cpad:w * cpad + n_cells] for w in range(NW)]
    scal = _tc_combine(ca, aggs, b_dis, b_ang)

    tabs = list(scal) + [cell_size[:, 0], cell_size[:, 1]]
    cols = _sc_gather(tabs, fathers, sons)
    edge_dis, edge_angle = _tc_readout(cols)
    return (edge_dis, edge_angle)
